# baseline (speedup 1.0000x reference)
import functools

import jax
import jax.numpy as jnp
from jax import lax
from jax.experimental import pallas as pl
from jax.experimental.pallas import tpu as pltpu
from jax.experimental.pallas import tpu_sc as plsc

L = 16
NB_U = 8
N = 4096
B = 4
BIG = 3.0e38

SC_PRED = 2048
CHUNK = SC_PRED // 32
XG = CHUNK // L
MV = N // L
MSL = N // 16

TC_NBLK = 1024
TC_JB = N // TC_NBLK
TC_J0 = SC_PRED // TC_NBLK
TC_STEPS = (TC_JB - TC_J0) + (B - 1) * TC_JB


def _bf16_round(v):
    u = plsc.bitcast(v, jnp.int32)
    r = (u + 0x7FFF + ((u >> 16) & 1)) & jnp.int32(-65536)
    return plsc.bitcast(r, jnp.float32)


def _sc_part(xs, ys):
    mesh = plsc.VectorSubcoreMesh(core_axis_name="c", subcore_axis_name="s",
                                  num_cores=2, num_subcores=16)

    @functools.partial(
        pl.kernel,
        out_type=(jax.ShapeDtypeStruct((2 * N,), jnp.float32),
                  jax.ShapeDtypeStruct((2 * L,), jnp.float32)),
        mesh=mesh,
        compiler_params=pltpu.CompilerParams(needs_layout_passes=False),
        scratch_types=[
            pltpu.VMEM((3 * CHUNK,), jnp.float32),
            pltpu.VMEM((CHUNK,), jnp.float32),
            pltpu.VMEM((3 * N,), jnp.float32),
            pltpu.VMEM((N,), jnp.float32),
            pltpu.VMEM((3 * N,), jnp.float32),
            pltpu.VMEM((N,), jnp.float32),
            pltpu.VMEM((16 * MSL,), jnp.float32),
            pltpu.VMEM((16 * L,), jnp.float32),
            pltpu.VMEM((L,), jnp.float32),
            pltpu.VMEM_SHARED((16 * N,), jnp.float32),
            pltpu.VMEM_SHARED((16 * L,), jnp.float32),
        ],
    )
    def k(xs_hbm, ys_hbm, cm_hbm, rs_hbm,
          xvr, x2h, yvr, y2h, ytmp, colmin, cbuf, st3, sbuf,
          cmin_sh, scal_sh):
        c = lax.axis_index("c")
        s = lax.axis_index("s")
        n0 = (c * 16 + s) * CHUNK

        for d in range(3):
            pltpu.sync_copy(xs_hbm.at[pl.ds(d * SC_PRED + n0, CHUNK)],
                            cbuf.at[pl.ds(d * CHUNK, CHUNK)])
            pltpu.sync_copy(ys_hbm.at[pl.ds(d * N, N)],
                            ytmp.at[pl.ds(d * N, N)])

        half = jnp.float32(0.5)
        lane = lax.iota(jnp.int32, L)

        def prep_x(j, _):
            o = j * L
            x0 = cbuf[pl.ds(o, L)]
            x1 = cbuf[pl.ds(CHUNK + o, L)]
            x2 = cbuf[pl.ds(2 * CHUNK + o, L)]
            xvr[pl.ds(o, L)] = _bf16_round(x0)
            xvr[pl.ds(CHUNK + o, L)] = _bf16_round(x1)
            xvr[pl.ds(2 * CHUNK + o, L)] = _bf16_round(x2)
            x2h[pl.ds(o, L)] = (x0 * x0 + x1 * x1 + x2 * x2) * half
            return 0
        lax.fori_loop(0, XG, prep_x, 0)

        def prep_y(j, _):
            o = j * L
            y0 = ytmp[pl.ds(o, L)]
            y1 = ytmp[pl.ds(N + o, L)]
            y2 = ytmp[pl.ds(2 * N + o, L)]
            yvr[pl.ds(o, L)] = _bf16_round(y0)
            yvr[pl.ds(N + o, L)] = _bf16_round(y1)
            yvr[pl.ds(2 * N + o, L)] = _bf16_round(y2)
            y2h[pl.ds(o, L)] = (y0 * y0 + y1 * y1 + y2 * y2) * half
            colmin[pl.ds(o, L)] = jnp.full((L,), BIG, jnp.float32)
            return 0
        lax.fori_loop(0, MV, prep_y, 0)
        zeros = jnp.zeros((L,), jnp.float32)

        def make_mstep(xb):
            def mstep(j, rms):
                o = j * L
                y0 = yvr[pl.ds(o, L)]
                y1 = yvr[pl.ds(N + o, L)]
                y2 = yvr[pl.ds(2 * N + o, L)]
                sv = y2h[pl.ds(o, L)]
                cv = colmin[pl.ds(o, L)]
                new = []
                d2s = []
                for u in range(NB_U):
                    cross = xb[u][0] * y0 + xb[u][1] * y1 + xb[u][2] * y2
                    d2 = (sv + xb[u][3]) - cross
                    d2s.append(d2)
                    new.append(jnp.minimum(rms[u], d2))
                m01 = jnp.minimum(d2s[0], d2s[1])
                m23 = jnp.minimum(d2s[2], d2s[3])
                m45 = jnp.minimum(d2s[4], d2s[5])
                m67 = jnp.minimum(d2s[6], d2s[7])
                m = jnp.minimum(jnp.minimum(m01, m23), jnp.minimum(m45, m67))
                colmin[pl.ds(o, L)] = jnp.minimum(cv, m)
                return tuple(new)
            return mstep

        def nblock(ng, rsacc):
            base = ng * L
            x0v = xvr[pl.ds(base, L)]
            x1v = xvr[pl.ds(CHUNK + base, L)]
            x2v = xvr[pl.ds(2 * CHUNK + base, L)]
            x2hv = x2h[pl.ds(base, L)]

            def half_block(h, rsacc):
                xb = [(lax.broadcast(x0v[h * NB_U + u], (L,)),
                       lax.broadcast(x1v[h * NB_U + u], (L,)),
                       lax.broadcast(x2v[h * NB_U + u], (L,)),
                       lax.broadcast(x2hv[h * NB_U + u], (L,)))
                      for u in range(NB_U)]
                rm0 = tuple(jnp.full((L,), BIG, jnp.float32)
                            for _ in range(NB_U))
                rms = lax.fori_loop(0, MV, make_mstep(xb), rm0)
                for u in range(NB_U):
                    rm = jnp.maximum(rms[u], 0.0)
                    rsacc = rsacc + lax.broadcast(jnp.min(rm), (L,))
                return rsacc

            rsacc = half_block(0, rsacc)
            rsacc = half_block(1, rsacc)
            return rsacc

        rsacc = lax.fori_loop(0, XG, nblock, zeros)

        pltpu.sync_copy(colmin, cmin_sh.at[pl.ds(s * N, N)])
        plsc.subcore_barrier()

        ms = s * MSL
        for i in range(16):
            pltpu.sync_copy(cmin_sh.at[pl.ds(i * N + ms, MSL)],
                            cbuf.at[pl.ds(i * MSL, MSL)])

        two = jnp.float32(2.0)

        def cstep(j, _):
            o = j * L
            vs = [cbuf[pl.ds(i * MSL + o, L)] for i in range(16)]
            for st in (8, 4, 2, 1):
                vs = [jnp.minimum(vs[i], vs[i + st]) for i in range(st)]
            colmin[pl.ds(o, L)] = vs[0] * two
            return 0
        lax.fori_loop(0, MSL // L, cstep, 0)
        pltpu.sync_copy(colmin.at[pl.ds(0, MSL)],
                        cm_hbm.at[pl.ds(c * N + ms, MSL)])

        pv = jnp.where(lane == 0, rsacc * two, zeros)
        sbuf[...] = pv
        pltpu.sync_copy(sbuf, scal_sh.at[pl.ds(s * L, L)])
        plsc.subcore_barrier()

        @pl.when(s == 0)
        def _():
            pltpu.sync_copy(scal_sh, st3)

            def rstep(i, acc):
                return acc + st3[pl.ds(i * L, L)]
            acc = lax.fori_loop(0, 16, rstep, zeros)
            sbuf[...] = acc
            pltpu.sync_copy(sbuf, rs_hbm.at[pl.ds(c * L, L)])

    return k(xs, ys)


def _tc_step(x_ref, y_ref, chams_ref, cm0_ref, y3v_scr, colmin_scr, rs_smem):
    i = pl.program_id(0)
    bj = i + TC_J0
    j = bj % TC_JB
    first = jnp.logical_or(i == 0, j == 0)
    last = j == TC_JB - 1

    @pl.when(first)
    def _():
        y3v_scr[...] = jnp.transpose(y_ref[0], (1, 0))

    xb = x_ref[0]
    y3 = y3v_scr[...]
    x2 = jnp.sum(xb * xb, axis=1, keepdims=True)
    y2 = (y3[0:1] * y3[0:1] + y3[1:2] * y3[1:2]
          + y3[2:3] * y3[2:3])
    xb2 = (xb + xb).astype(jnp.bfloat16)
    y3b = y3.astype(jnp.bfloat16)
    cross2 = lax.dot_general(xb2, y3b, (((1,), (0,)), ((), ())),
                             preferred_element_type=jnp.float32)
    d2 = (x2 + y2) - cross2

    rowmin = jnp.min(d2, axis=1)
    rsum = jnp.sum(jnp.maximum(rowmin, 0.0))
    bmin = jnp.min(d2, axis=0, keepdims=True)

    @pl.when(first)
    def _():
        rs_smem[0] = rsum
        colmin_scr[...] = bmin

    @pl.when(jnp.logical_not(first))
    def _():
        rs_smem[0] = rs_smem[0] + rsum
        colmin_scr[...] = jnp.minimum(colmin_scr[...], bmin)

    @pl.when(last)
    def _():
        lanes = lax.broadcasted_iota(jnp.int32, (8, 128), 1)
        csum = jnp.sum(jnp.maximum(colmin_scr[...], 0.0))
        chams_ref[...] = (jnp.where(lanes == 0, rs_smem[0], 0.0)
                          + jnp.where(lanes == 1, csum, 0.0))

    @pl.when(jnp.logical_and(last, i == TC_JB - TC_J0 - 1))
    def _():
        cm0_ref[...] = colmin_scr[...]


def _tc_part(x, y):
    def bmap(i):
        return (i + TC_J0) // TC_JB

    return pl.pallas_call(
        _tc_step,
        grid=(TC_STEPS,),
        in_specs=[
            pl.BlockSpec((1, TC_NBLK, 3), lambda i: (bmap(i), (i + TC_J0) % TC_JB, 0)),
            pl.BlockSpec((1, N, 3), lambda i: (bmap(i), 0, 0)),
        ],
        out_specs=[
            pl.BlockSpec((8, 128), lambda i: (bmap(i), 0)),
            pl.BlockSpec((1, N), lambda i: (0, 0)),
        ],
        out_shape=[
            jax.ShapeDtypeStruct((8 * B, 128), jnp.float32),
            jax.ShapeDtypeStruct((1, N), jnp.float32),
        ],
        scratch_shapes=[
            pltpu.VMEM((3, N), jnp.float32),
            pltpu.VMEM((1, N), jnp.float32),
            pltpu.SMEM((1,), jnp.float32),
        ],
    )(x, y)


def _merge_step(sc_cm_ref, sc_rs_ref, tc_ch_ref, tc_cm0_ref, fb_ref, out_ref):
    cm0 = jnp.minimum(jnp.minimum(sc_cm_ref[pl.ds(0, N)], sc_cm_ref[pl.ds(N, N)]),
                      tc_cm0_ref[0])
    csum0 = jnp.sum(jnp.maximum(cm0, 0.0))
    rs0 = sc_rs_ref[0] + sc_rs_ref[L] + tc_ch_ref[0, 0]
    inv_n = jnp.float32(1.0 / N)
    cham0 = (rs0 + csum0) * inv_n
    cham123 = (tc_ch_ref[8, 0] + tc_ch_ref[8, 1]
               + tc_ch_ref[16, 0] + tc_ch_ref[16, 1]
               + tc_ch_ref[24, 0] + tc_ch_ref[24, 1]) * inv_n
    dist = (cham0 + cham123) * jnp.float32(1.0 / B)
    lanes_b = lax.broadcasted_iota(jnp.int32, (1, 128), 1)
    rate = jnp.sum(jnp.where(lanes_b < B, fb_ref[...], 0.0)) \
        * jnp.float32(1.0 / B)
    loss = dist + rate
    lanes = lax.broadcasted_iota(jnp.int32, (1, 128), 1)
    out_ref[...] = (jnp.where(lanes == 0, loss, 0.0)
                    + jnp.where(lanes == 1, dist, 0.0)
                    + jnp.where(lanes == 2, rate, 0.0))


def _merge(sc_cm, sc_rs, tc_ch, tc_cm0, fb):
    return pl.pallas_call(
        _merge_step,
        out_shape=jax.ShapeDtypeStruct((1, 128), jnp.float32),
        in_specs=[
            pl.BlockSpec((2 * N,), lambda: (0,)),
            pl.BlockSpec((2 * L,), lambda: (0,)),
            pl.BlockSpec((8 * B, 128), lambda: (0, 0)),
            pl.BlockSpec((1, N), lambda: (0, 0)),
            pl.BlockSpec((1, 128), lambda: (0, 0)),
        ],
        out_specs=pl.BlockSpec((1, 128), lambda: (0, 0)),
    )(sc_cm, sc_rs, tc_ch, tc_cm0, fb)


@jax.jit
def kernel(pc_pred, pc_target, fbpp):
    xs0 = jnp.transpose(pc_pred[0, :SC_PRED], (1, 0)).reshape(-1)
    ys0 = jnp.transpose(pc_target[0], (1, 0)).reshape(-1)

    sc_cm, sc_rs = _sc_part(xs0, ys0)
    tc_ch, tc_cm0 = _tc_part(pc_pred, pc_target)

    fb = jnp.pad(fbpp, (0, 128 - B)).reshape(1, 128)
    out = _merge(sc_cm, sc_rs, tc_ch, tc_cm0, fb)
    loss = out[0, 0]
    dist = out[0, 1]
    rate = out[0, 2]
    return (loss, dist, rate)


# speedup vs baseline: 1.0423x; 1.0423x over previous
import functools

import jax
import jax.numpy as jnp
from jax import lax
from jax.experimental import pallas as pl
from jax.experimental.pallas import tpu as pltpu
from jax.experimental.pallas import tpu_sc as plsc

L = 16
NB_U = 8
N = 4096
B = 4
BIG = 3.0e38

SC_PRED = 2048
CHUNK = SC_PRED // 32
XG = CHUNK // L
MV = N // L
MSL = N // 16

TC_NBLK = 1024
TC_JB = N // TC_NBLK
TC_J0 = SC_PRED // TC_NBLK
TC_STEPS = (TC_JB - TC_J0) + (B - 1) * TC_JB


def _bf16_round(v):
    u = plsc.bitcast(v, jnp.int32)
    r = (u + 0x7FFF + ((u >> 16) & 1)) & jnp.int32(-65536)
    return plsc.bitcast(r, jnp.float32)


def _sc_part(xs, ys):
    mesh = plsc.VectorSubcoreMesh(core_axis_name="c", subcore_axis_name="s",
                                  num_cores=2, num_subcores=16)

    @functools.partial(
        pl.kernel,
        out_type=(jax.ShapeDtypeStruct((2 * N,), jnp.float32),
                  jax.ShapeDtypeStruct((2 * L,), jnp.float32)),
        mesh=mesh,
        compiler_params=pltpu.CompilerParams(needs_layout_passes=False),
        scratch_types=[
            pltpu.VMEM((3 * CHUNK,), jnp.float32),
            pltpu.VMEM((CHUNK,), jnp.float32),
            pltpu.VMEM((3 * N,), jnp.float32),
            pltpu.VMEM((N,), jnp.float32),
            pltpu.VMEM((3 * N,), jnp.float32),
            pltpu.VMEM((N,), jnp.float32),
            pltpu.VMEM((16 * MSL,), jnp.float32),
            pltpu.VMEM((16 * L,), jnp.float32),
            pltpu.VMEM((L,), jnp.float32),
            pltpu.VMEM_SHARED((16 * N,), jnp.float32),
            pltpu.VMEM_SHARED((16 * L,), jnp.float32),
        ],
    )
    def k(xs_hbm, ys_hbm, cm_hbm, rs_hbm,
          xvr, x2h, yvr, y2h, ytmp, colmin, cbuf, st3, sbuf,
          cmin_sh, scal_sh):
        c = lax.axis_index("c")
        s = lax.axis_index("s")
        n0 = (c * 16 + s) * CHUNK

        for d in range(3):
            pltpu.sync_copy(xs_hbm.at[pl.ds(d * SC_PRED + n0, CHUNK)],
                            cbuf.at[pl.ds(d * CHUNK, CHUNK)])
            pltpu.sync_copy(ys_hbm.at[pl.ds(d * N, N)],
                            ytmp.at[pl.ds(d * N, N)])

        half = jnp.float32(0.5)
        lane = lax.iota(jnp.int32, L)

        def prep_x(j, _):
            o = j * L
            x0 = cbuf[pl.ds(o, L)]
            x1 = cbuf[pl.ds(CHUNK + o, L)]
            x2 = cbuf[pl.ds(2 * CHUNK + o, L)]
            xvr[pl.ds(o, L)] = _bf16_round(x0)
            xvr[pl.ds(CHUNK + o, L)] = _bf16_round(x1)
            xvr[pl.ds(2 * CHUNK + o, L)] = _bf16_round(x2)
            x2h[pl.ds(o, L)] = (x0 * x0 + x1 * x1 + x2 * x2) * half
            return 0
        lax.fori_loop(0, XG, prep_x, 0)

        def prep_y(j, _):
            o = j * L
            y0 = ytmp[pl.ds(o, L)]
            y1 = ytmp[pl.ds(N + o, L)]
            y2 = ytmp[pl.ds(2 * N + o, L)]
            yvr[pl.ds(o, L)] = _bf16_round(y0)
            yvr[pl.ds(N + o, L)] = _bf16_round(y1)
            yvr[pl.ds(2 * N + o, L)] = _bf16_round(y2)
            y2h[pl.ds(o, L)] = (y0 * y0 + y1 * y1 + y2 * y2) * half
            colmin[pl.ds(o, L)] = jnp.full((L,), BIG, jnp.float32)
            return 0
        lax.fori_loop(0, MV, prep_y, 0)
        zeros = jnp.zeros((L,), jnp.float32)

        def make_mstep(xb):
            def mstep(j, rms):
                o = j * L
                y0 = yvr[pl.ds(o, L)]
                y1 = yvr[pl.ds(N + o, L)]
                y2 = yvr[pl.ds(2 * N + o, L)]
                sv = y2h[pl.ds(o, L)]
                cv = colmin[pl.ds(o, L)]
                new = []
                d2s = []
                for u in range(NB_U):
                    cross = xb[u][0] * y0 + xb[u][1] * y1 + xb[u][2] * y2
                    d2 = (sv + xb[u][3]) - cross
                    d2s.append(d2)
                    new.append(jnp.minimum(rms[u], d2))
                m01 = jnp.minimum(d2s[0], d2s[1])
                m23 = jnp.minimum(d2s[2], d2s[3])
                m45 = jnp.minimum(d2s[4], d2s[5])
                m67 = jnp.minimum(d2s[6], d2s[7])
                m = jnp.minimum(jnp.minimum(m01, m23), jnp.minimum(m45, m67))
                colmin[pl.ds(o, L)] = jnp.minimum(cv, m)
                return tuple(new)
            return mstep

        def nblock(ng, rsacc):
            base = ng * L
            x0v = xvr[pl.ds(base, L)]
            x1v = xvr[pl.ds(CHUNK + base, L)]
            x2v = xvr[pl.ds(2 * CHUNK + base, L)]
            x2hv = x2h[pl.ds(base, L)]

            def half_block(h, rsacc):
                xb = [(lax.broadcast(x0v[h * NB_U + u], (L,)),
                       lax.broadcast(x1v[h * NB_U + u], (L,)),
                       lax.broadcast(x2v[h * NB_U + u], (L,)),
                       lax.broadcast(x2hv[h * NB_U + u], (L,)))
                      for u in range(NB_U)]
                rm0 = tuple(jnp.full((L,), BIG, jnp.float32)
                            for _ in range(NB_U))
                rms = lax.fori_loop(0, MV, make_mstep(xb), rm0)
                for u in range(NB_U):
                    rm = jnp.maximum(rms[u], 0.0)
                    rsacc = rsacc + lax.broadcast(jnp.min(rm), (L,))
                return rsacc

            rsacc = half_block(0, rsacc)
            rsacc = half_block(1, rsacc)
            return rsacc

        rsacc = lax.fori_loop(0, XG, nblock, zeros)

        pltpu.sync_copy(colmin, cmin_sh.at[pl.ds(s * N, N)])
        plsc.subcore_barrier()

        ms = s * MSL
        for i in range(16):
            pltpu.sync_copy(cmin_sh.at[pl.ds(i * N + ms, MSL)],
                            cbuf.at[pl.ds(i * MSL, MSL)])

        two = jnp.float32(2.0)

        def cstep(j, _):
            o = j * L
            vs = [cbuf[pl.ds(i * MSL + o, L)] for i in range(16)]
            for st in (8, 4, 2, 1):
                vs = [jnp.minimum(vs[i], vs[i + st]) for i in range(st)]
            colmin[pl.ds(o, L)] = vs[0] * two
            return 0
        lax.fori_loop(0, MSL // L, cstep, 0)
        pltpu.sync_copy(colmin.at[pl.ds(0, MSL)],
                        cm_hbm.at[pl.ds(c * N + ms, MSL)])

        pv = jnp.where(lane == 0, rsacc * two, zeros)
        sbuf[...] = pv
        pltpu.sync_copy(sbuf, scal_sh.at[pl.ds(s * L, L)])
        plsc.subcore_barrier()

        @pl.when(s == 0)
        def _():
            pltpu.sync_copy(scal_sh, st3)

            def rstep(i, acc):
                return acc + st3[pl.ds(i * L, L)]
            acc = lax.fori_loop(0, 16, rstep, zeros)
            sbuf[...] = acc
            pltpu.sync_copy(sbuf, rs_hbm.at[pl.ds(c * L, L)])

    return k(xs, ys)


def _tc_step(x_ref, y_ref, chams_ref, cm0_ref, colmin_scr, rs_smem):
    i = pl.program_id(0)
    bj = i + TC_J0
    j = bj % TC_JB
    first = jnp.logical_or(i == 0, j == 0)
    last = j == TC_JB - 1

    xb = x_ref[0]
    y3 = y_ref[0]
    x2 = jnp.sum(xb * xb, axis=1, keepdims=True)
    y2 = (y3[0:1] * y3[0:1] + y3[1:2] * y3[1:2]
          + y3[2:3] * y3[2:3])
    xb2 = (xb + xb).astype(jnp.bfloat16)
    y3b = y3.astype(jnp.bfloat16)
    cross2 = lax.dot_general(xb2, y3b, (((1,), (0,)), ((), ())),
                             preferred_element_type=jnp.float32)
    d2 = (x2 + y2) - cross2

    rowmin = jnp.min(d2, axis=1)
    rsum = jnp.sum(jnp.maximum(rowmin, 0.0))
    bmin = jnp.min(d2, axis=0, keepdims=True)

    @pl.when(first)
    def _():
        rs_smem[0] = rsum
        colmin_scr[...] = bmin

    @pl.when(jnp.logical_not(first))
    def _():
        rs_smem[0] = rs_smem[0] + rsum
        colmin_scr[...] = jnp.minimum(colmin_scr[...], bmin)

    @pl.when(last)
    def _():
        lanes = lax.broadcasted_iota(jnp.int32, (8, 128), 1)
        csum = jnp.sum(jnp.maximum(colmin_scr[...], 0.0))
        chams_ref[...] = (jnp.where(lanes == 0, rs_smem[0], 0.0)
                          + jnp.where(lanes == 1, csum, 0.0))

    @pl.when(jnp.logical_and(last, i == TC_JB - TC_J0 - 1))
    def _():
        cm0_ref[...] = colmin_scr[...]


def _tc_part(x, y3):
    def bmap(i):
        return (i + TC_J0) // TC_JB

    return pl.pallas_call(
        _tc_step,
        grid=(TC_STEPS,),
        in_specs=[
            pl.BlockSpec((1, TC_NBLK, 3), lambda i: (bmap(i), (i + TC_J0) % TC_JB, 0)),
            pl.BlockSpec((1, 3, N), lambda i: (bmap(i), 0, 0)),
        ],
        out_specs=[
            pl.BlockSpec((8, 128), lambda i: (bmap(i), 0)),
            pl.BlockSpec((1, N), lambda i: (0, 0)),
        ],
        out_shape=[
            jax.ShapeDtypeStruct((8 * B, 128), jnp.float32),
            jax.ShapeDtypeStruct((1, N), jnp.float32),
        ],
        scratch_shapes=[
            pltpu.VMEM((1, N), jnp.float32),
            pltpu.SMEM((1,), jnp.float32),
        ],
    )(x, y3)


def _merge_step(sc_cm_ref, sc_rs_ref, tc_ch_ref, tc_cm0_ref, fb_ref, out_ref):
    cm0 = jnp.minimum(jnp.minimum(sc_cm_ref[pl.ds(0, N)], sc_cm_ref[pl.ds(N, N)]),
                      tc_cm0_ref[0])
    csum0 = jnp.sum(jnp.maximum(cm0, 0.0))
    rs0 = sc_rs_ref[0] + sc_rs_ref[L] + tc_ch_ref[0, 0]
    inv_n = jnp.float32(1.0 / N)
    cham0 = (rs0 + csum0) * inv_n
    cham123 = (tc_ch_ref[8, 0] + tc_ch_ref[8, 1]
               + tc_ch_ref[16, 0] + tc_ch_ref[16, 1]
               + tc_ch_ref[24, 0] + tc_ch_ref[24, 1]) * inv_n
    dist = (cham0 + cham123) * jnp.float32(1.0 / B)
    lanes_b = lax.broadcasted_iota(jnp.int32, (1, 128), 1)
    rate = jnp.sum(jnp.where(lanes_b < B, fb_ref[...], 0.0)) \
        * jnp.float32(1.0 / B)
    loss = dist + rate
    lanes = lax.broadcasted_iota(jnp.int32, (1, 128), 1)
    out_ref[...] = (jnp.where(lanes == 0, loss, 0.0)
                    + jnp.where(lanes == 1, dist, 0.0)
                    + jnp.where(lanes == 2, rate, 0.0))


def _merge(sc_cm, sc_rs, tc_ch, tc_cm0, fb):
    return pl.pallas_call(
        _merge_step,
        out_shape=jax.ShapeDtypeStruct((1, 128), jnp.float32),
        in_specs=[
            pl.BlockSpec((2 * N,), lambda: (0,)),
            pl.BlockSpec((2 * L,), lambda: (0,)),
            pl.BlockSpec((8 * B, 128), lambda: (0, 0)),
            pl.BlockSpec((1, N), lambda: (0, 0)),
            pl.BlockSpec((1, 128), lambda: (0, 0)),
        ],
        out_specs=pl.BlockSpec((1, 128), lambda: (0, 0)),
    )(sc_cm, sc_rs, tc_ch, tc_cm0, fb)


@jax.jit
def kernel(pc_pred, pc_target, fbpp):
    xs0 = jnp.transpose(pc_pred[0, :SC_PRED], (1, 0)).reshape(-1)
    ys0 = jnp.transpose(pc_target[0], (1, 0)).reshape(-1)

    sc_cm, sc_rs = _sc_part(xs0, ys0)
    tc_ch, tc_cm0 = _tc_part(pc_pred, jnp.transpose(pc_target, (0, 2, 1)))

    fb = jnp.pad(fbpp, (0, 128 - B)).reshape(1, 128)
    out = _merge(sc_cm, sc_rs, tc_ch, tc_cm0, fb)
    loss = out[0, 0]
    dist = out[0, 1]
    rate = out[0, 2]
    return (loss, dist, rate)
